# Initial kernel scaffold; baseline (speedup 1.0000x reference)
import jax
import jax.numpy as jnp
from jax.experimental import pallas as pl


def kernel(edge_attr, params, x, node_depth, edge_index, batch):
    raise NotImplementedError("write your pallas kernel here")


# speedup vs baseline: 1.0138x; 1.0138x over previous
import functools

import jax
import jax.numpy as jnp
from jax import lax
from jax.experimental import pallas as pl
from jax.experimental.pallas import tpu as pltpu
from jax.experimental.pallas import tpu_sc as plsc

N = 50000
E = 800000
EMB = 64
B = 64
NUM_LAYERS = 3
MAX_DEPTH = 20
EPS = 1e-5

HALF = 32
CHUNK = 128
NS = 16
E_PAD = 800768
E_SUB = E_PAD // NS
NCHUNKS = E_SUB // CHUNK
N_PAD = 51200
ROWS_SUB = N_PAD // NS
NROWCP = ROWS_SUB // CHUNK


def _edge_sc_call(hxr, row2f, colf, eef, nbf):
    mesh = plsc.VectorSubcoreMesh(core_axis_name="c", subcore_axis_name="s")

    @functools.partial(
        pl.kernel,
        mesh=mesh,
        compiler_params=pltpu.CompilerParams(use_tc_tiling_on_sc=False),
        out_type=jax.ShapeDtypeStruct((2 * N_PAD, HALF), jnp.float32),
        scratch_types=[
            pltpu.VMEM((1, CHUNK), jnp.int32),
            pltpu.VMEM((1, CHUNK), jnp.int32),
            pltpu.VMEM((CHUNK, HALF), jnp.float32),
            pltpu.VMEM((CHUNK, HALF), jnp.float32),
            pltpu.VMEM((CHUNK, HALF), jnp.float32),
            pltpu.VMEM_SHARED((N_PAD, HALF), jnp.float32),
            pltpu.SemaphoreType.DMA,
        ],
    )
    def ek(hxr_h, row2_h, col_h, ee_h, nb_h, out_h,
           idxg, idxs, rows_v, ee_v, nb_v, acc, sem):
        c = lax.axis_index("c")
        s = lax.axis_index("s")
        zero16 = jnp.zeros((16,), jnp.float32)

        @pl.loop(0, CHUNK)
        def _zb(i):
            ee_v[i, pl.ds(0, 16)] = zero16
            ee_v[i, pl.ds(16, 16)] = zero16

        base0 = s * ROWS_SUB

        @pl.loop(0, NROWCP)
        def _za(i):
            pltpu.sync_copy(ee_v, acc.at[pl.ds(base0 + i * CHUNK, CHUNK)])

        plsc.subcore_barrier()

        ebase = s * E_SUB

        @pl.loop(0, E_SUB)
        def _edges(e):
            j = e // CHUNK
            within = e % CHUNK

            @pl.when(within == 0)
            def _fetch():
                off = ebase + j * CHUNK
                pltpu.sync_copy(row2_h.at[pl.ds(c * E_PAD + off, CHUNK)],
                                idxg.at[0])
                pltpu.sync_copy(col_h.at[pl.ds(off, CHUNK)], idxs.at[0])
                pltpu.async_copy(hxr_h.at[idxg.at[0]], rows_v, sem).wait()
                pltpu.sync_copy(ee_h.at[pl.ds(c * E_PAD + off, CHUNK)], ee_v)
                pltpu.sync_copy(nb_h.at[pl.ds(off, CHUNK)], nb_v)

            for half in range(2):
                sl = pl.ds(half * 16, 16)
                rows_v[within, sl] = (
                    jnp.maximum(rows_v[within, sl] + ee_v[within, sl], 0.0)
                    * nb_v[within, sl])

            @pl.when(within == CHUNK - 1)
            def _scat():
                pltpu.sync_copy(rows_v, acc.at[idxs.at[0]], add=True)

        plsc.subcore_barrier()

        @pl.loop(0, NROWCP)
        def _wb(i):
            off = base0 + i * CHUNK
            pltpu.sync_copy(acc.at[pl.ds(off, CHUNK)],
                            out_h.at[pl.ds(c * N_PAD + off, CHUNK)])

    return ek(hxr, row2f, colf, eef, nbf)


def _batchnorm(h, gamma, beta):
    mean = h.mean(axis=0)
    var = ((h - mean) ** 2).mean(axis=0)
    return (h - mean) / jnp.sqrt(var + EPS) * gamma + beta


def kernel(edge_attr, params, x, node_depth, edge_index, batch):
    row, col = edge_index[0], edge_index[1]

    deg = jax.ops.segment_sum(jnp.ones((E,), jnp.float32), row,
                              num_segments=N) + 1.0
    dis = deg ** -0.5
    norm = dis[row] * dis[col]

    rowp = jnp.pad(row, (0, E_PAD - E)).astype(jnp.int32)
    row2f = jnp.concatenate([2 * rowp, 2 * rowp + 1])
    colf = jnp.pad(col, (0, E_PAD - E),
                   constant_values=N_PAD - 1).astype(jnp.int32)
    normp = jnp.pad(norm, (0, E_PAD - E))
    nbf = jnp.broadcast_to(normp[:, None], (E_PAD, HALF))

    depth = jnp.minimum(node_depth, MAX_DEPTH)
    h = (params['type_emb'][x[:, 0]]
         + params['attr_emb'][x[:, 1]]
         + params['depth_emb'][depth])
    vn = params['vn_emb'][jnp.zeros((B,), dtype=jnp.int32)]

    for layer in range(NUM_LAYERS):
        p = params['convs'][layer]
        h = h + vn[batch]
        hx = h @ p['lin_W'] + p['lin_b']

        ee = edge_attr @ p['edge_W'] + p['edge_b']
        eep = jnp.pad(ee, ((0, E_PAD - E), (0, 0)))
        eef = jnp.concatenate([eep[:, :HALF], eep[:, HALF:]], axis=0)
        hxr = hx.reshape(2 * N, HALF)

        out2 = _edge_sc_call(hxr, row2f, colf, eef, nbf)
        agg = jnp.concatenate([out2[:N], out2[N_PAD:N_PAD + N]], axis=1)

        h = agg + jax.nn.relu(hx + p['bias']) / deg[:, None]
        h = _batchnorm(h, params['bn_gamma'][layer], params['bn_beta'][layer])
        if layer == NUM_LAYERS - 1:
            break
        h = jax.nn.relu(h)
        vn = vn + jax.ops.segment_sum(h, batch, num_segments=B)
        mp = params['mlps'][layer]
        v = vn @ mp['W1'] + mp['b1']
        v = _batchnorm(v, mp['g1'], mp['bt1'])
        v = jax.nn.relu(v)
        vn = v @ mp['W2'] + mp['b2']
    return h


# speedup vs baseline: 1.1119x; 1.0967x over previous
import functools

import jax
import jax.numpy as jnp
from jax import lax
from jax.experimental import pallas as pl
from jax.experimental.pallas import tpu as pltpu
from jax.experimental.pallas import tpu_sc as plsc

N = 50000
E = 800000
EMB = 64
B = 64
NUM_LAYERS = 3
MAX_DEPTH = 20
EPS = 1e-5

HALF = 32
CHUNK = 128
NS = 16
E_PAD = 800768
E_SUB = E_PAD // NS
NCHUNKS = E_SUB // CHUNK
N_PAD = 51200
ROWS_SUB = N_PAD // NS
NROWCP = ROWS_SUB // CHUNK


def _edge_sc_call(hxr, row2f, colf, eef, nbf):
    mesh = plsc.VectorSubcoreMesh(core_axis_name="c", subcore_axis_name="s")

    @functools.partial(
        pl.kernel,
        mesh=mesh,
        compiler_params=pltpu.CompilerParams(use_tc_tiling_on_sc=False),
        out_type=jax.ShapeDtypeStruct((2 * N_PAD, HALF), jnp.float32),
        scratch_types=[
            pltpu.VMEM((1, CHUNK), jnp.int32),
            pltpu.VMEM((1, CHUNK), jnp.int32),
            pltpu.VMEM((CHUNK, HALF), jnp.float32),
            pltpu.VMEM((CHUNK, HALF), jnp.float32),
            pltpu.VMEM((CHUNK, HALF), jnp.float32),
            pltpu.VMEM_SHARED((N_PAD, HALF), jnp.float32),
            pltpu.SemaphoreType.DMA,
            pltpu.SemaphoreType.DMA,
        ],
    )
    def ek(hxr_h, row2_h, col_h, ee_h, nb_h, out_h,
           idxg, idxs, rows_v, ee_v, nb_v, acc, sem, sem2):
        c = lax.axis_index("c")
        s = lax.axis_index("s")
        zero16 = jnp.zeros((16,), jnp.float32)

        @pl.loop(0, CHUNK)
        def _zb(i):
            ee_v[i, pl.ds(0, 16)] = zero16
            ee_v[i, pl.ds(16, 16)] = zero16

        base0 = s * ROWS_SUB

        @pl.loop(0, NROWCP)
        def _za(i):
            pltpu.sync_copy(ee_v, acc.at[pl.ds(base0 + i * CHUNK, CHUNK)])

        plsc.subcore_barrier()

        ebase = s * E_SUB

        @pl.loop(0, E_SUB)
        def _edges(e):
            j = e // CHUNK
            within = e % CHUNK

            @pl.when(within == 0)
            def _fetch():
                off = ebase + j * CHUNK
                cp_col = pltpu.async_copy(col_h.at[pl.ds(off, CHUNK)],
                                          idxs.at[0], sem2)
                cp_ee = pltpu.async_copy(ee_h.at[pl.ds(c * E_PAD + off, CHUNK)],
                                         ee_v, sem2)
                cp_nb = pltpu.async_copy(nb_h.at[pl.ds(off, CHUNK)],
                                         nb_v, sem2)
                pltpu.sync_copy(row2_h.at[pl.ds(c * E_PAD + off, CHUNK)],
                                idxg.at[0])
                cp_g = pltpu.async_copy(hxr_h.at[idxg.at[0]], rows_v, sem)
                cp_g.wait()
                cp_col.wait()
                cp_ee.wait()
                cp_nb.wait()

            for half in range(2):
                sl = pl.ds(half * 16, 16)
                rows_v[within, sl] = (
                    jnp.maximum(rows_v[within, sl] + ee_v[within, sl], 0.0)
                    * nb_v[within, sl])

            @pl.when(within == CHUNK - 1)
            def _scat():
                pltpu.sync_copy(rows_v, acc.at[idxs.at[0]], add=True)

        plsc.subcore_barrier()

        @pl.loop(0, NROWCP)
        def _wb(i):
            off = base0 + i * CHUNK
            pltpu.sync_copy(acc.at[pl.ds(off, CHUNK)],
                            out_h.at[pl.ds(c * N_PAD + off, CHUNK)])

    return ek(hxr, row2f, colf, eef, nbf)


def _batchnorm(h, gamma, beta):
    mean = h.mean(axis=0)
    var = ((h - mean) ** 2).mean(axis=0)
    return (h - mean) / jnp.sqrt(var + EPS) * gamma + beta


def kernel(edge_attr, params, x, node_depth, edge_index, batch):
    row, col = edge_index[0], edge_index[1]

    deg = jax.ops.segment_sum(jnp.ones((E,), jnp.float32), row,
                              num_segments=N) + 1.0
    dis = deg ** -0.5
    norm = dis[row] * dis[col]

    rowp = jnp.pad(row, (0, E_PAD - E)).astype(jnp.int32)
    row2f = jnp.concatenate([2 * rowp, 2 * rowp + 1])
    colf = jnp.pad(col, (0, E_PAD - E),
                   constant_values=N_PAD - 1).astype(jnp.int32)
    normp = jnp.pad(norm, (0, E_PAD - E))
    nbf = jnp.broadcast_to(normp[:, None], (E_PAD, HALF))

    depth = jnp.minimum(node_depth, MAX_DEPTH)
    h = (params['type_emb'][x[:, 0]]
         + params['attr_emb'][x[:, 1]]
         + params['depth_emb'][depth])
    vn = params['vn_emb'][jnp.zeros((B,), dtype=jnp.int32)]

    for layer in range(NUM_LAYERS):
        p = params['convs'][layer]
        h = h + vn[batch]
        hx = h @ p['lin_W'] + p['lin_b']

        ee = edge_attr @ p['edge_W'] + p['edge_b']
        eep = jnp.pad(ee, ((0, E_PAD - E), (0, 0)))
        eef = jnp.concatenate([eep[:, :HALF], eep[:, HALF:]], axis=0)
        hxr = hx.reshape(2 * N, HALF)

        out2 = _edge_sc_call(hxr, row2f, colf, eef, nbf)
        agg = jnp.concatenate([out2[:N], out2[N_PAD:N_PAD + N]], axis=1)

        h = agg + jax.nn.relu(hx + p['bias']) / deg[:, None]
        h = _batchnorm(h, params['bn_gamma'][layer], params['bn_beta'][layer])
        if layer == NUM_LAYERS - 1:
            break
        h = jax.nn.relu(h)
        vn = vn + jax.ops.segment_sum(h, batch, num_segments=B)
        mp = params['mlps'][layer]
        v = vn @ mp['W1'] + mp['b1']
        v = _batchnorm(v, mp['g1'], mp['bt1'])
        v = jax.nn.relu(v)
        vn = v @ mp['W2'] + mp['b2']
    return h
